# baseline (speedup 1.0000x reference)
import functools

import jax
import jax.numpy as jnp
from jax import lax
from jax.experimental import pallas as pl
from jax.experimental.pallas import tpu as pltpu
from jax.experimental.pallas import tpu_sc as plsc

B = 4
C = 192
S = 48
K = 4
H = 224
W = 224
HW = H * W
NCH = 56
T = HW // NCH
QUADS = (B * S) // K
GRPS = NCH // K
NITEMS = QUADS * GRPS

NC = 2
NS = 16
NW = NC * NS
ITEMS_PER_W = NITEMS // NW


def _sc_kernel(x_hbm, idx_hbm, out_hbm, idx_v, gidx, gbuf, obuf, sem):
    wid = lax.axis_index("s") * NC + lax.axis_index("c")

    pltpu.sync_copy(idx_hbm, idx_v)

    def item_body(it, _):
        item = wid * ITEMS_PER_W + it
        quad = item // GRPS
        grp = item % GRPS
        b = quad // (QUADS // B)
        sbase = (quad % (QUADS // B)) * K
        t0 = grp * K

        ch = idx_v[pl.ds(sbase * K, 16)]
        base_rows = (b * C + ch) * NCH
        for tt in range(K):
            gidx[tt, :] = base_rows + (t0 + tt)

        cps = [pltpu.async_copy(x_hbm.at[gidx.at[tt]], gbuf.at[tt], sem)
               for tt in range(K)]
        for cp in cps:
            cp.wait()

        def body(k, _):
            sl = pl.ds(k * 16, 16)
            for q in range(K):
                for tt in range(K):
                    m0 = jnp.maximum(gbuf[tt, 4 * q, sl],
                                     gbuf[tt, 4 * q + 1, sl])
                    m1 = jnp.maximum(gbuf[tt, 4 * q + 2, sl],
                                     gbuf[tt, 4 * q + 3, sl])
                    obuf[q * K + tt, sl] = jnp.maximum(m0, m1)
            return 0

        lax.fori_loop(0, T // 16, body, 0)

        wcps = []
        for q in range(K):
            orow = (b * S + sbase + q) * NCH + t0
            wcps.append(pltpu.async_copy(
                obuf.at[pl.ds(q * K, K)], out_hbm.at[pl.ds(orow, K)], sem))
        for cp in wcps:
            cp.wait()
        return 0

    lax.fori_loop(0, ITEMS_PER_W, item_body, 0)


@jax.jit
def _fused_multi_pool(x_flat, idx_flat):
    mesh = plsc.VectorSubcoreMesh(core_axis_name="c", subcore_axis_name="s")
    run = functools.partial(
        pl.kernel,
        out_type=jax.ShapeDtypeStruct((B * S * NCH, T), jnp.float32),
        mesh=mesh,
        scratch_types=[
            pltpu.VMEM((S * K,), jnp.int32),
            pltpu.VMEM((K, 16), jnp.int32),
            pltpu.VMEM((K, 16, T), jnp.float32),
            pltpu.VMEM((16, T), jnp.float32),
            pltpu.SemaphoreType.DMA,
        ],
    )(_sc_kernel)
    return run(x_flat, idx_flat)


def kernel(input, channel_idx_sets):
    x_flat = input.reshape(B * C * NCH, T)
    out = _fused_multi_pool(x_flat, channel_idx_sets.reshape(S * K))
    return out.reshape(B, S, H, W)


# speedup vs baseline: 1.0342x; 1.0342x over previous
import functools

import jax
import jax.numpy as jnp
from jax import lax
from jax.experimental import pallas as pl
from jax.experimental.pallas import tpu as pltpu
from jax.experimental.pallas import tpu_sc as plsc

B = 4
C = 192
S = 48
K = 4
H = 224
W = 224
HW = H * W
NCH = 56
T = HW // NCH
HNCH = NCH // 2
QUADS = (B * S) // K
SQ = S // K
GRPS = HNCH
NITEMS = QUADS * GRPS

NC = 2
NS = 16
NW = NC * NS
IPW = NITEMS // NW
NPAIR = IPW // 2


def _sc_kernel(x_hbm, idx_hbm, out_hbm, idx_v, gidx, gbuf, obuf,
               semg0, semg1, sems0, sems1):
    wid = lax.axis_index("s") * NC + lax.axis_index("c")
    base_item = wid * IPW
    semg = (semg0, semg1)
    sems = (sems0, sems1)

    pltpu.sync_copy(idx_hbm, idx_v)

    def split(item):
        quad = item // GRPS
        grp = item % GRPS
        b = quad // SQ
        sbase = (quad % SQ) * K
        return b, sbase, grp

    def gather_item(item, slot):
        b, sbase, grp = split(item)
        ch = idx_v[pl.ds(sbase * K, 16)]
        gidx[slot, :] = (b * C + ch) * HNCH + grp
        pltpu.async_copy(x_hbm.at[gidx.at[slot]], gbuf.at[slot], semg[slot])

    def drain_gather(slot):
        pltpu.make_async_copy(
            x_hbm.at[pl.ds(0, 16)], gbuf.at[slot], semg[slot]).wait()

    def drain_store(slot):
        pltpu.make_async_copy(
            obuf.at[slot], out_hbm.at[pl.ds(0, K)], sems[slot]).wait()

    def compute_store(item, slot):
        b, sbase, grp = split(item)

        def body(k, _):
            sl = pl.ds(k * 16, 16)
            for q in range(K):
                for tt in range(2):
                    m0 = jnp.maximum(gbuf[slot, 4 * q, tt, sl],
                                     gbuf[slot, 4 * q + 1, tt, sl])
                    m1 = jnp.maximum(gbuf[slot, 4 * q + 2, tt, sl],
                                     gbuf[slot, 4 * q + 3, tt, sl])
                    obuf[slot, q, tt, sl] = jnp.maximum(m0, m1)
            return 0

        lax.fori_loop(0, T // 16, body, 0)
        for q in range(K):
            orow = (b * S + sbase + q) * HNCH + grp
            pltpu.async_copy(obuf.at[slot, q], out_hbm.at[orow], sems[slot])

    gather_item(base_item, 0)

    def step(i, _):
        a = base_item + 2 * i
        gather_item(a + 1, 1)
        drain_gather(0)
        pl.when(i > 0)(lambda: drain_store(0))
        compute_store(a, 0)
        pl.when(i < NPAIR - 1)(lambda: gather_item(a + 2, 0))
        drain_gather(1)
        pl.when(i > 0)(lambda: drain_store(1))
        compute_store(a + 1, 1)
        return 0

    lax.fori_loop(0, NPAIR, step, 0)
    drain_store(0)
    drain_store(1)


@jax.jit
def _fused_multi_pool(x3, idx_flat):
    mesh = plsc.VectorSubcoreMesh(core_axis_name="c", subcore_axis_name="s")
    run = functools.partial(
        pl.kernel,
        out_type=jax.ShapeDtypeStruct((B * S * HNCH, 2, T), jnp.float32),
        mesh=mesh,
        scratch_types=[
            pltpu.VMEM((S * K,), jnp.int32),
            pltpu.VMEM((2, 16), jnp.int32),
            pltpu.VMEM((2, 16, 2, T), jnp.float32),
            pltpu.VMEM((2, K, 2, T), jnp.float32),
            pltpu.SemaphoreType.DMA,
            pltpu.SemaphoreType.DMA,
            pltpu.SemaphoreType.DMA,
            pltpu.SemaphoreType.DMA,
        ],
    )(_sc_kernel)
    return run(x3, idx_flat)


def kernel(input, channel_idx_sets):
    x3 = input.reshape(B * C * HNCH, 2, T)
    out = _fused_multi_pool(x3, channel_idx_sets.reshape(S * K))
    return out.reshape(B, S, H, W)


# speedup vs baseline: 1.0495x; 1.0149x over previous
import functools

import jax
import jax.numpy as jnp
from jax import lax
from jax.experimental import pallas as pl
from jax.experimental.pallas import tpu as pltpu
from jax.experimental.pallas import tpu_sc as plsc

B = 4
C = 192
S = 48
K = 4
H = 224
W = 224
HW = H * W
NCH = 56
T = HW // NCH
HNCH = NCH // 2
QUADS = (B * S) // K
SQ = S // K
GRPS = HNCH
NITEMS = QUADS * GRPS

NC = 2
NS = 16
NW = NC * NS
IPW = NITEMS // NW
NBUF = 3
NSTEP = IPW // NBUF


def _sc_kernel(x_hbm, idx_hbm, out_hbm, idx_v, gidx, gbuf, obuf,
               semg0, semg1, semg2, sems0, sems1, sems2):
    wid = lax.axis_index("s") * NC + lax.axis_index("c")
    base_item = wid * IPW
    semg = (semg0, semg1, semg2)
    sems = (sems0, sems1, sems2)

    pltpu.sync_copy(idx_hbm, idx_v)

    def split(item):
        quad = item // GRPS
        grp = item % GRPS
        b = quad // SQ
        sbase = (quad % SQ) * K
        return b, sbase, grp

    def gather_item(item, slot):
        b, sbase, grp = split(item)
        ch = idx_v[pl.ds(sbase * K, 16)]
        gidx[slot, :] = (b * C + ch) * HNCH + grp
        pltpu.async_copy(x_hbm.at[gidx.at[slot]], gbuf.at[slot], semg[slot])

    def drain_gather(slot):
        pltpu.make_async_copy(
            x_hbm.at[pl.ds(0, 16)], gbuf.at[slot], semg[slot]).wait()

    def drain_store(slot):
        pltpu.make_async_copy(
            obuf.at[slot], out_hbm.at[pl.ds(0, K)], sems[slot]).wait()

    def compute_store(item, slot):
        b, sbase, grp = split(item)

        def body(k, _):
            sl = pl.ds(k * 16, 16)
            for q in range(K):
                for tt in range(2):
                    m0 = jnp.maximum(gbuf[slot, 4 * q, tt, sl],
                                     gbuf[slot, 4 * q + 1, tt, sl])
                    m1 = jnp.maximum(gbuf[slot, 4 * q + 2, tt, sl],
                                     gbuf[slot, 4 * q + 3, tt, sl])
                    obuf[slot, q, tt, sl] = jnp.maximum(m0, m1)
            return 0

        lax.fori_loop(0, T // 16, body, 0)
        for q in range(K):
            orow = (b * S + sbase + q) * HNCH + grp
            pltpu.async_copy(obuf.at[slot, q], out_hbm.at[orow], sems[slot])

    gather_item(base_item, 0)
    gather_item(base_item + 1, 1)

    def step(i, _):
        for u in range(NBUF):
            n = i * NBUF + u
            su = (u + 2) % NBUF
            pl.when(n + 2 < IPW)(
                lambda n=n, su=su: gather_item(base_item + n + 2, su))
            drain_gather(u)
            pl.when(n >= NBUF)(lambda u=u: drain_store(u))
            compute_store(base_item + n, u)
        return 0

    lax.fori_loop(0, NSTEP, step, 0)
    for u in range(NBUF):
        drain_store(u)


@jax.jit
def _fused_multi_pool(x3, idx_flat):
    mesh = plsc.VectorSubcoreMesh(core_axis_name="c", subcore_axis_name="s")
    run = functools.partial(
        pl.kernel,
        out_type=jax.ShapeDtypeStruct((B * S * HNCH, 2, T), jnp.float32),
        mesh=mesh,
        scratch_types=[
            pltpu.VMEM((S * K,), jnp.int32),
            pltpu.VMEM((NBUF, 16), jnp.int32),
            pltpu.VMEM((NBUF, 16, 2, T), jnp.float32),
            pltpu.VMEM((NBUF, K, 2, T), jnp.float32),
            pltpu.SemaphoreType.DMA,
            pltpu.SemaphoreType.DMA,
            pltpu.SemaphoreType.DMA,
            pltpu.SemaphoreType.DMA,
            pltpu.SemaphoreType.DMA,
            pltpu.SemaphoreType.DMA,
        ],
    )(_sc_kernel)
    return run(x3, idx_flat)


def kernel(input, channel_idx_sets):
    x3 = input.reshape(B * C * HNCH, 2, T)
    out = _fused_multi_pool(x3, channel_idx_sets.reshape(S * K))
    return out.reshape(B, S, H, W)
